# baseline (device time: 56301 ns/iter reference)
import jax
import jax.numpy as jnp
from jax import lax
from jax.experimental import pallas as pl
from jax.experimental.pallas import tpu as pltpu

N_DEV = 8


def kernel(x, W1, W2):
    m_per, d_model = x.shape
    half = m_per // 2
    quart = half // 2

    def body(x_ref, W1_ref, W2_ref, out_ref, xgR, xgL, rsR, rsL, accR, accL,
             w1b, w2b,
             agR_ssem, agR_rsem, agL_ssem, agL_rsem,
             rsR_ssem, rsR_rsem, rsL_ssem, rsL_rsem):
        my = lax.axis_index("i")
        left = lax.rem(my - 1 + N_DEV, N_DEV)
        right = lax.rem(my + 1, N_DEV)

        barrier_sem = pltpu.get_barrier_semaphore()
        for nbr in (left, right):
            pl.semaphore_signal(
                barrier_sem, inc=1,
                device_id=(nbr,), device_id_type=pl.DeviceIdType.MESH,
            )
        pl.semaphore_wait(barrier_sem, 2)

        def contribution(xblk):
            h = jnp.dot(xblk, w1b[...], preferred_element_type=jnp.float32)
            h = h * jax.nn.sigmoid(h)
            return jnp.dot(h.astype(jnp.bfloat16), w2b[...],
                           preferred_element_type=jnp.float32)

        streams = (
            ("R", 0), ("L", 0), ("R", 1), ("L", 1),
        )

        def refs(dirn):
            if dirn == "R":
                return xgR, rsR, accR, agR_ssem, agR_rsem, rsR_ssem, rsR_rsem, right
            return xgL, rsL, accL, agL_ssem, agL_rsem, rsL_ssem, rsL_rsem, left

        def ag_blk(dirn, h):
            return (my - h) % N_DEV if dirn == "R" else (my + h) % N_DEV

        def rs_chunk(dirn, s):
            return (my - 1 - s) % N_DEV if dirn == "R" else (my + 1 + s) % N_DEV

        def ag_rdma(dirn, sub, h):
            xg, _, _, ssem, rsem, _, _, peer = refs(dirn)
            blk = ag_blk(dirn, h)
            return pltpu.make_async_remote_copy(
                src_ref=xg.at[blk, sub], dst_ref=xg.at[blk, sub],
                send_sem=ssem.at[h, sub], recv_sem=rsem.at[h, sub],
                device_id=(peer,), device_id_type=pl.DeviceIdType.MESH,
            )

        def rs_rdma(dirn, sub, s):
            _, rs, acc, _, _, ssem, rsem, peer = refs(dirn)
            return pltpu.make_async_remote_copy(
                src_ref=acc.at[s % 2, sub], dst_ref=rs.at[s, sub],
                send_sem=ssem.at[s, sub], recv_sem=rsem.at[s, sub],
                device_id=(peer,), device_id_type=pl.DeviceIdType.MESH,
            )

        xgR[my, 0] = x_ref[:quart, :].astype(jnp.bfloat16)
        xgR[my, 1] = x_ref[quart:half, :].astype(jnp.bfloat16)
        xgL[my, 0] = x_ref[half:half + quart, :].astype(jnp.bfloat16)
        xgL[my, 1] = x_ref[half + quart:, :].astype(jnp.bfloat16)
        for dirn, sub in streams:
            ag_rdma(dirn, sub, 0).start()

        w1b[...] = W1_ref[...].astype(jnp.bfloat16)
        w2b[...] = W2_ref[...].astype(jnp.bfloat16)

        for s in range(N_DEV - 1):
            for dirn, sub in streams:
                xg, rs, acc, _, _, _, _, _ = refs(dirn)
                ag_rdma(dirn, sub, s).wait_recv()
                if s < N_DEV - 2:
                    ag_rdma(dirn, sub, s + 1).start()
                part = contribution(xg[rs_chunk(dirn, s), sub])
                if s > 1:
                    rs_rdma(dirn, sub, s - 2).wait_send()
                if s > 0:
                    rs_rdma(dirn, sub, s - 1).wait_recv()
                    part = part + rs[s - 1, sub].astype(jnp.float32)
                acc[s % 2, sub] = part.astype(jnp.bfloat16)
                rs_rdma(dirn, sub, s).start()
            if s == 0:
                out_ref[:quart, :] = contribution(xgR[my, 0])
                out_ref[quart:half, :] = contribution(xgR[my, 1])
                out_ref[half:half + quart, :] = contribution(xgL[my, 0])
                out_ref[half + quart:, :] = contribution(xgL[my, 1])

        row0 = {("R", 0): 0, ("R", 1): quart, ("L", 0): half,
                ("L", 1): half + quart}
        for dirn, sub in streams:
            _, rs, _, _, _, _, _, _ = refs(dirn)
            rs_rdma(dirn, sub, N_DEV - 2).wait_recv()
            lo = row0[(dirn, sub)]
            out_ref[lo:lo + quart, :] += rs[N_DEV - 2, sub].astype(jnp.float32)

        for dirn, sub in streams:
            for h in range(N_DEV - 1):
                ag_rdma(dirn, sub, h).wait_send()
            for s in (N_DEV - 3, N_DEV - 2):
                rs_rdma(dirn, sub, s).wait_send()

    return pl.pallas_call(
        body,
        out_shape=jax.ShapeDtypeStruct((m_per, d_model), jnp.float32),
        in_specs=[
            pl.BlockSpec(memory_space=pltpu.VMEM),
            pl.BlockSpec(memory_space=pltpu.VMEM),
            pl.BlockSpec(memory_space=pltpu.VMEM),
        ],
        out_specs=pl.BlockSpec(memory_space=pltpu.VMEM),
        scratch_shapes=[
            pltpu.VMEM((N_DEV, 2, quart, d_model), jnp.bfloat16),
            pltpu.VMEM((N_DEV, 2, quart, d_model), jnp.bfloat16),
            pltpu.VMEM((N_DEV - 1, 2, quart, d_model), jnp.bfloat16),
            pltpu.VMEM((N_DEV - 1, 2, quart, d_model), jnp.bfloat16),
            pltpu.VMEM((2, 2, quart, d_model), jnp.bfloat16),
            pltpu.VMEM((2, 2, quart, d_model), jnp.bfloat16),
            pltpu.VMEM(W1.shape, jnp.bfloat16),
            pltpu.VMEM(W2.shape, jnp.bfloat16),
            pltpu.SemaphoreType.DMA((N_DEV - 1, 2)),
            pltpu.SemaphoreType.DMA((N_DEV - 1, 2)),
            pltpu.SemaphoreType.DMA((N_DEV - 1, 2)),
            pltpu.SemaphoreType.DMA((N_DEV - 1, 2)),
            pltpu.SemaphoreType.DMA((N_DEV - 1, 2)),
            pltpu.SemaphoreType.DMA((N_DEV - 1, 2)),
            pltpu.SemaphoreType.DMA((N_DEV - 1, 2)),
            pltpu.SemaphoreType.DMA((N_DEV - 1, 2)),
        ],
        compiler_params=pltpu.CompilerParams(collective_id=0),
    )(x.astype(jnp.bfloat16), W1, W2)
